# Initial kernel scaffold; baseline (speedup 1.0000x reference)
import jax
import jax.numpy as jnp
from jax.experimental import pallas as pl


def kernel(x, edge_index, batch, W1, a1s, a1d, b1, W2, a2s, a2d, b2, W3, a3s, a3d, b3, Wfc, bfc):
    raise NotImplementedError("write your pallas kernel here")


# speedup vs baseline: 14.1361x; 14.1361x over previous
import jax
import jax.numpy as jnp
from jax import lax
from jax.experimental import pallas as pl
from jax.experimental.pallas import tpu as pltpu
from jax.experimental.pallas import tpu_sc as plsc

N = 10000
NPAD = 10240
E = 320000
H = 8
C = 64
D = 512
NG = 64
NCLS = 16

NCORE = 2
NSUB = 16
NTILES = NCORE * NSUB
EB = 128
NBPT = 79
EPT = NBPT * EB
EPAD = EPT * NTILES
RB = 256
NRB = NPAD // RB
RPT = NPAD // NSUB

_F32 = jnp.float32


def _mm_tail(h, h8_ref, t1_ref, t2_ref, am_ref):
    t = jnp.dot(h, am_ref[...], preferred_element_type=_F32)
    t1_ref[...] = t[:, :16]
    t2_ref[...] = t[:, 16:]
    for hh in range(H):
        h8_ref[hh] = h[:, C * hh:C * (hh + 1)]


def _layer1_body(x_ref, w_ref, am_ref, h8_ref, t1_ref, t2_ref):
    h = jnp.dot(x_ref[...], w_ref[...], preferred_element_type=_F32)
    _mm_tail(h, h8_ref, t1_ref, t2_ref, am_ref)


def _prep_body(ow_ref, den_ref, b_ref, w_ref, am_ref,
               h8_ref, t1_ref, t2_ref, xin_ref):
    den = den_ref[0] + den_ref[1]
    rec = 1.0 / (den + 1e-16)
    for hh in range(H):
        o = ow_ref[0, hh] + ow_ref[1, hh]
        sl = slice(C * hh, C * (hh + 1))
        xin_ref[:, sl] = jnp.maximum(
            o * rec[:, hh:hh + 1] + b_ref[0, sl], 0.0)
    h = jnp.dot(xin_ref[...], w_ref[...], preferred_element_type=_F32)
    _mm_tail(h, h8_ref, t1_ref, t2_ref, am_ref)


def _layer_out_shapes():
    return (
        jax.ShapeDtypeStruct((H, NPAD, C), _F32),
        jax.ShapeDtypeStruct((NPAD, 16), _F32),
        jax.ShapeDtypeStruct((NPAD, 16), _F32),
    )


def _layer_out_specs():
    return (
        pl.BlockSpec((H, RB, C), lambda i: (0, i, 0)),
        pl.BlockSpec((RB, 16), lambda i: (i, 0)),
        pl.BlockSpec((RB, 16), lambda i: (i, 0)),
    )


def _tc_layer1(xp, w, am):
    k = xp.shape[1]
    return pl.pallas_call(
        _layer1_body,
        grid=(NRB,),
        in_specs=[
            pl.BlockSpec((RB, k), lambda i: (i, 0)),
            pl.BlockSpec((k, D), lambda i: (0, 0)),
            pl.BlockSpec((D, 32), lambda i: (0, 0)),
        ],
        out_specs=_layer_out_specs(),
        out_shape=_layer_out_shapes(),
    )(xp, w, am)


def _tc_prep_layer(ow, den, b, w, am):
    return pl.pallas_call(
        _prep_body,
        grid=(NRB,),
        in_specs=[
            pl.BlockSpec((2, H, RB, C), lambda i: (0, 0, i, 0)),
            pl.BlockSpec((2, RB, 16), lambda i: (0, i, 0)),
            pl.BlockSpec((1, D), lambda i: (0, 0)),
            pl.BlockSpec((D, D), lambda i: (0, 0)),
            pl.BlockSpec((D, 32), lambda i: (0, 0)),
        ],
        out_specs=_layer_out_specs(),
        out_shape=_layer_out_shapes(),
        scratch_shapes=[pltpu.VMEM((RB, D), _F32)],
    )(ow, den, b, w, am)


def _final_body(ow_ref, den_ref, b3_ref, batch_ref, wfc_ref, bfc_ref,
                out_ref, pooled_ref, cnt_ref):
    i = pl.program_id(0)

    @pl.when(i == 0)
    def _():
        pooled_ref[...] = jnp.zeros_like(pooled_ref)
        cnt_ref[...] = jnp.zeros_like(cnt_ref)

    den = den_ref[0] + den_ref[1]
    rec = 1.0 / (den + 1e-16)
    acc = jnp.zeros((RB, C), _F32)
    for hh in range(H):
        o = ow_ref[0, hh] + ow_ref[1, hh]
        acc = acc + o * rec[:, hh:hh + 1]
    h3 = jnp.maximum(acc / 8.0 + b3_ref[0], 0.0)

    rows = i * RB + lax.broadcasted_iota(jnp.int32, (NG, RB), 1)
    valid = rows < N
    grp = lax.broadcasted_iota(jnp.int32, (NG, RB), 0)
    onehot = jnp.where((batch_ref[0, 0][None, :] == grp) & valid, 1.0, 0.0)
    pooled_ref[...] += jnp.dot(onehot, h3, preferred_element_type=_F32)
    cnt_ref[...] += jnp.broadcast_to(
        jnp.sum(onehot, axis=1, keepdims=True), (NG, C))

    @pl.when(i == NRB - 1)
    def _():
        pd = pooled_ref[...] / jnp.maximum(cnt_ref[...], 1.0)
        logits = jnp.dot(pd, wfc_ref[...], preferred_element_type=_F32)
        logits = logits + bfc_ref[0]
        col = lax.broadcasted_iota(jnp.int32, (NG, 128), 1)
        masked = jnp.where(col < NCLS, logits, -1e30)
        m = jnp.max(masked, axis=1, keepdims=True)
        ex = jnp.where(col < NCLS, jnp.exp(masked - m), 0.0)
        lse = m + jnp.log(jnp.sum(ex, axis=1, keepdims=True))
        out_ref[...] = masked - lse


def _tc_final(ow, den, b3, batch2, wfcp, bfcp):
    return pl.pallas_call(
        _final_body,
        grid=(NRB,),
        in_specs=[
            pl.BlockSpec((2, H, RB, C), lambda i: (0, 0, i, 0)),
            pl.BlockSpec((2, RB, 16), lambda i: (0, i, 0)),
            pl.BlockSpec((1, C), lambda i: (0, 0)),
            pl.BlockSpec((1, 1, RB), lambda i: (i, 0, 0)),
            pl.BlockSpec((C, 128), lambda i: (0, 0)),
            pl.BlockSpec((1, 128), lambda i: (0, 0)),
        ],
        out_specs=pl.BlockSpec((NG, 128), lambda i: (0, 0)),
        out_shape=jax.ShapeDtypeStruct((NG, 128), _F32),
        scratch_shapes=[pltpu.VMEM((NG, C), _F32), pltpu.VMEM((NG, C), _F32)],
    )(ow, den, b3, batch2, wfcp, bfcp)


def _sc_body(t1h, t2h, h8h, srch, dsth, owh, denh, exoh,
             srcb, dstb, ts, td, exb, rows, hidx, zb, zbd, oacc, dacc):
    cid = lax.axis_index("c")
    sid = lax.axis_index("s")
    tile = cid * NSUB + sid
    row0 = tile * NBPT
    zero16 = jnp.zeros((16,), _F32)

    @pl.loop(0, EB)
    def _(r):
        for j in range(4):
            zb[r, pl.ds(16 * j, 16)] = zero16
        zbd[r, :] = zero16

    pltpu.sync_copy(srch.at[tile], srcb)
    pltpu.sync_copy(dsth.at[tile], dstb)

    for z in range(RPT // EB):
        pltpu.sync_copy(zbd, dacc.at[pl.ds(sid * RPT + z * EB, EB)])
    plsc.subcore_barrier()

    @pl.loop(0, NBPT)
    def _(b):
        pltpu.sync_copy(t1h.at[srcb.at[b]], ts)
        pltpu.sync_copy(t2h.at[dstb.at[b]], td)

        @pl.loop(0, EB)
        def _(r):
            v = ts[r, :] + td[r, :]
            v = jnp.where(v >= 0.0, v, 0.2 * v)
            exb[r, :] = jnp.exp(v)

        pltpu.sync_copy(exb, exoh.at[pl.ds((row0 + b) * EB, EB)])
        pltpu.sync_copy(exb, dacc.at[dstb.at[b]], add=True)

    plsc.subcore_barrier()
    pltpu.sync_copy(dacc.at[pl.ds(sid * RPT, RPT)],
                    denh.at[pl.ds(cid * NPAD + sid * RPT, RPT)])

    for hh in range(H):
        for z in range(RPT // EB):
            pltpu.sync_copy(zb, oacc.at[pl.ds(sid * RPT + z * EB, EB)])
        plsc.subcore_barrier()

        @pl.loop(0, NBPT)
        def _(b):
            @pl.loop(0, 8)
            def _(j):
                hidx[pl.ds(16 * j, 16)] = (
                    srcb[b, pl.ds(16 * j, 16)] + hh * NPAD)

            pltpu.sync_copy(h8h.at[hidx], rows)
            pltpu.sync_copy(exoh.at[pl.ds((row0 + b) * EB, EB)], exb)

            @pl.loop(0, EB)
            def _(r):
                exr = exb[r, :]
                s = jnp.full(
                    (16,), jnp.squeeze(lax.slice(exr, (hh,), (hh + 1,))),
                    _F32)
                for j in range(4):
                    sl = pl.ds(16 * j, 16)
                    rows[r, sl] = rows[r, sl] * s

            pltpu.sync_copy(rows, oacc.at[dstb.at[b]], add=True)

        plsc.subcore_barrier()
        pltpu.sync_copy(
            oacc.at[pl.ds(sid * RPT, RPT)],
            owh.at[pl.ds((cid * H + hh) * NPAD + sid * RPT, RPT)])
        plsc.subcore_barrier()


def _sc_gat(t1, t2, h8f, src2, dst2):
    mesh = plsc.VectorSubcoreMesh(core_axis_name="c", subcore_axis_name="s")
    run = pl.kernel(
        _sc_body,
        out_type=(
            jax.ShapeDtypeStruct((2 * H * NPAD, C), _F32),
            jax.ShapeDtypeStruct((2 * NPAD, 16), _F32),
            jax.ShapeDtypeStruct((EPAD, 16), _F32),
        ),
        mesh=mesh,
        compiler_params=pltpu.CompilerParams(use_tc_tiling_on_sc=False),
        scratch_types=(
            pltpu.VMEM((NBPT, EB), jnp.int32),
            pltpu.VMEM((NBPT, EB), jnp.int32),
            pltpu.VMEM((EB, 16), _F32),
            pltpu.VMEM((EB, 16), _F32),
            pltpu.VMEM((EB, 16), _F32),
            pltpu.VMEM((EB, C), _F32),
            pltpu.VMEM((EB,), jnp.int32),
            pltpu.VMEM((EB, C), _F32),
            pltpu.VMEM((EB, 16), _F32),
            pltpu.VMEM_SHARED((NPAD, C), _F32),
            pltpu.VMEM_SHARED((NPAD, 16), _F32),
        ),
    )
    return run(t1, t2, h8f, src2, dst2)


def _build_am(a_s, a_d):
    eye8 = jnp.eye(8, dtype=_F32)
    ms = (a_s[:, :, None] * eye8[:, None, :]).reshape(D, 8)
    md = (a_d[:, :, None] * eye8[:, None, :]).reshape(D, 8)
    ms2 = jnp.concatenate([ms, ms], axis=1)
    md2 = jnp.concatenate([md, md], axis=1)
    return jnp.concatenate([ms2, md2], axis=1)


def kernel(x, edge_index, batch, W1, a1s, a1d, b1, W2, a2s, a2d, b2,
           W3, a3s, a3d, b3, Wfc, bfc):
    xp = jnp.zeros((NPAD, x.shape[1]), _F32).at[:N].set(x)
    pad = jnp.full((EPAD - E,), N, jnp.int32)
    src2 = jnp.concatenate([edge_index[0].astype(jnp.int32), pad]
                           ).reshape(NTILES, NBPT, EB)
    dst2 = jnp.concatenate([edge_index[1].astype(jnp.int32), pad]
                           ).reshape(NTILES, NBPT, EB)
    batch2 = jnp.zeros((NPAD,), jnp.int32).at[:N].set(
        batch.astype(jnp.int32)).reshape(NRB, 1, RB)

    wfcp = jnp.zeros((C, 128), _F32).at[:, :NCLS].set(Wfc)
    bfcp = jnp.zeros((1, 128), _F32).at[0, :NCLS].set(bfc)

    h8, t1, t2 = _tc_layer1(xp, W1, _build_am(a1s, a1d))
    ow, den, _ = _sc_gat(t1, t2, h8.reshape(H * NPAD, C), src2, dst2)

    h8, t1, t2 = _tc_prep_layer(ow.reshape(2, H, NPAD, C),
                                den.reshape(2, NPAD, 16),
                                b1.reshape(1, D), W2, _build_am(a2s, a2d))
    ow, den, _ = _sc_gat(t1, t2, h8.reshape(H * NPAD, C), src2, dst2)

    h8, t1, t2 = _tc_prep_layer(ow.reshape(2, H, NPAD, C),
                                den.reshape(2, NPAD, 16),
                                b2.reshape(1, D), W3, _build_am(a3s, a3d))
    ow, den, _ = _sc_gat(t1, t2, h8.reshape(H * NPAD, C), src2, dst2)

    out = _tc_final(ow.reshape(2, H, NPAD, C), den.reshape(2, NPAD, 16),
                    b3.reshape(1, C), batch2, wfcp, bfcp)
    return out[:, :NCLS]


# speedup vs baseline: 15.8159x; 1.1188x over previous
import jax
import jax.numpy as jnp
from jax import lax
from jax.experimental import pallas as pl
from jax.experimental.pallas import tpu as pltpu
from jax.experimental.pallas import tpu_sc as plsc

N = 10000
NPAD = 10240
E = 320000
H = 8
C = 64
D = 512
NG = 64
NCLS = 16

NCORE = 2
NSUB = 16
NTILES = NCORE * NSUB
EB = 128
NBPT = 80
EPT = NBPT * EB
EPAD = EPT * NTILES
RB = 256
NRB = NPAD // RB
RPT = NPAD // NSUB

_F32 = jnp.float32


def _mm_tail(h, h8_ref, t1_ref, t2_ref, am_ref):
    t = jnp.dot(h, am_ref[...], preferred_element_type=_F32)
    t1_ref[...] = t[:, :16]
    t2_ref[...] = t[:, 16:]
    for hh in range(H):
        h8_ref[hh] = h[:, C * hh:C * (hh + 1)]


def _layer1_body(x_ref, w_ref, am_ref, h8_ref, t1_ref, t2_ref):
    h = jnp.dot(x_ref[...], w_ref[...], preferred_element_type=_F32)
    _mm_tail(h, h8_ref, t1_ref, t2_ref, am_ref)


def _prep_body(ow_ref, den_ref, b_ref, w_ref, am_ref,
               h8_ref, t1_ref, t2_ref, xin_ref):
    den = den_ref[0] + den_ref[1]
    rec = 1.0 / (den + 1e-16)
    for hh in range(H):
        o = ow_ref[0, hh] + ow_ref[1, hh]
        sl = slice(C * hh, C * (hh + 1))
        xin_ref[:, sl] = jnp.maximum(
            o * rec[:, hh:hh + 1] + b_ref[0, sl], 0.0)
    h = jnp.dot(xin_ref[...], w_ref[...], preferred_element_type=_F32)
    _mm_tail(h, h8_ref, t1_ref, t2_ref, am_ref)


def _layer_out_shapes():
    return (
        jax.ShapeDtypeStruct((H, NPAD, C), _F32),
        jax.ShapeDtypeStruct((NPAD, 16), _F32),
        jax.ShapeDtypeStruct((NPAD, 16), _F32),
    )


def _layer_out_specs():
    return (
        pl.BlockSpec((H, RB, C), lambda i: (0, i, 0)),
        pl.BlockSpec((RB, 16), lambda i: (i, 0)),
        pl.BlockSpec((RB, 16), lambda i: (i, 0)),
    )


def _tc_layer1(xp, w, am):
    k = xp.shape[1]
    return pl.pallas_call(
        _layer1_body,
        grid=(NRB,),
        in_specs=[
            pl.BlockSpec((RB, k), lambda i: (i, 0)),
            pl.BlockSpec((k, D), lambda i: (0, 0)),
            pl.BlockSpec((D, 32), lambda i: (0, 0)),
        ],
        out_specs=_layer_out_specs(),
        out_shape=_layer_out_shapes(),
    )(xp, w, am)


def _tc_prep_layer(ow, den, b, w, am):
    return pl.pallas_call(
        _prep_body,
        grid=(NRB,),
        in_specs=[
            pl.BlockSpec((2, H, RB, C), lambda i: (0, 0, i, 0)),
            pl.BlockSpec((2, RB, 16), lambda i: (0, i, 0)),
            pl.BlockSpec((1, D), lambda i: (0, 0)),
            pl.BlockSpec((D, D), lambda i: (0, 0)),
            pl.BlockSpec((D, 32), lambda i: (0, 0)),
        ],
        out_specs=_layer_out_specs(),
        out_shape=_layer_out_shapes(),
        scratch_shapes=[pltpu.VMEM((RB, D), _F32)],
    )(ow, den, b, w, am)


def _final_body(ow_ref, den_ref, b3_ref, batch_ref, wfc_ref, bfc_ref,
                out_ref, pooled_ref, cnt_ref):
    i = pl.program_id(0)

    @pl.when(i == 0)
    def _():
        pooled_ref[...] = jnp.zeros_like(pooled_ref)
        cnt_ref[...] = jnp.zeros_like(cnt_ref)

    den = den_ref[0] + den_ref[1]
    rec = 1.0 / (den + 1e-16)
    acc = jnp.zeros((RB, C), _F32)
    for hh in range(H):
        o = ow_ref[0, hh] + ow_ref[1, hh]
        acc = acc + o * rec[:, hh:hh + 1]
    h3 = jnp.maximum(acc / 8.0 + b3_ref[0], 0.0)

    rows = i * RB + lax.broadcasted_iota(jnp.int32, (NG, RB), 1)
    valid = rows < N
    grp = lax.broadcasted_iota(jnp.int32, (NG, RB), 0)
    onehot = jnp.where((batch_ref[0, 0][None, :] == grp) & valid, 1.0, 0.0)
    pooled_ref[...] += jnp.dot(onehot, h3, preferred_element_type=_F32)
    cnt_ref[...] += jnp.broadcast_to(
        jnp.sum(onehot, axis=1, keepdims=True), (NG, C))

    @pl.when(i == NRB - 1)
    def _():
        pd = pooled_ref[...] / jnp.maximum(cnt_ref[...], 1.0)
        logits = jnp.dot(pd, wfc_ref[...], preferred_element_type=_F32)
        logits = logits + bfc_ref[0]
        col = lax.broadcasted_iota(jnp.int32, (NG, 128), 1)
        masked = jnp.where(col < NCLS, logits, -1e30)
        m = jnp.max(masked, axis=1, keepdims=True)
        ex = jnp.where(col < NCLS, jnp.exp(masked - m), 0.0)
        lse = m + jnp.log(jnp.sum(ex, axis=1, keepdims=True))
        out_ref[...] = masked - lse


def _tc_final(ow, den, b3, batch2, wfcp, bfcp):
    return pl.pallas_call(
        _final_body,
        grid=(NRB,),
        in_specs=[
            pl.BlockSpec((2, H, RB, C), lambda i: (0, 0, i, 0)),
            pl.BlockSpec((2, RB, 16), lambda i: (0, i, 0)),
            pl.BlockSpec((1, C), lambda i: (0, 0)),
            pl.BlockSpec((1, 1, RB), lambda i: (i, 0, 0)),
            pl.BlockSpec((C, 128), lambda i: (0, 0)),
            pl.BlockSpec((1, 128), lambda i: (0, 0)),
        ],
        out_specs=pl.BlockSpec((NG, 128), lambda i: (0, 0)),
        out_shape=jax.ShapeDtypeStruct((NG, 128), _F32),
        scratch_shapes=[pltpu.VMEM((NG, C), _F32), pltpu.VMEM((NG, C), _F32)],
    )(ow, den, b3, batch2, wfcp, bfcp)


def _sc_body(t1h, t2h, h8h, srch, dsth, owh, denh, exoh,
             srcb, dstb, ts0, ts1, td0, td1, exb0, exb1, rows0, rows1,
             hidx0, hidx1, zb, zbd, oacc, dacc, gsem0, gsem1):
    cid = lax.axis_index("c")
    sid = lax.axis_index("s")
    tile = cid * NSUB + sid
    row0 = tile * NBPT
    zero16 = jnp.zeros((16,), _F32)
    tsl, tdl = (ts0, ts1), (td0, td1)
    exl, rwl = (exb0, exb1), (rows0, rows1)
    hxl, gsl = (hidx0, hidx1), (gsem0, gsem1)

    @pl.loop(0, EB, unroll=4)
    def _(r):
        for j in range(4):
            zb[r, pl.ds(16 * j, 16)] = zero16
        zbd[r, :] = zero16

    pltpu.sync_copy(srch.at[tile], srcb)
    pltpu.sync_copy(dsth.at[tile], dstb)

    for z in range(RPT // EB):
        pltpu.sync_copy(zbd, dacc.at[pl.ds(sid * RPT + z * EB, EB)])
    plsc.subcore_barrier()

    pltpu.async_copy(t1h.at[srcb.at[0]], ts0, gsem0)
    pltpu.async_copy(t2h.at[dstb.at[0]], td0, gsem0)

    @pl.loop(0, NBPT, step=2)
    def _(base):
        for i in range(2):
            b = base + i
            o = 1 - i
            pltpu.make_async_copy(t1h.at[srcb.at[b]], tsl[i], gsl[i]).wait()
            pltpu.make_async_copy(t2h.at[dstb.at[b]], tdl[i], gsl[i]).wait()

            @pl.when(b + 1 < NBPT)
            def _():
                pltpu.async_copy(t1h.at[srcb.at[b + 1]], tsl[o], gsl[o])
                pltpu.async_copy(t2h.at[dstb.at[b + 1]], tdl[o], gsl[o])

            ts, td, exb = tsl[i], tdl[i], exl[i]

            @pl.loop(0, EB, unroll=4)
            def _(r):
                v = ts[r, :] + td[r, :]
                v = jnp.where(v >= 0.0, v, 0.2 * v)
                exb[r, :] = jnp.exp(v)

            pltpu.sync_copy(exb, exoh.at[pl.ds((row0 + b) * EB, EB)])
            pltpu.sync_copy(exb, dacc.at[dstb.at[b]], add=True)

    plsc.subcore_barrier()
    pltpu.sync_copy(dacc.at[pl.ds(sid * RPT, RPT)],
                    denh.at[pl.ds(cid * NPAD + sid * RPT, RPT)])

    for hh in range(H):
        for z in range(RPT // EB):
            pltpu.sync_copy(zb, oacc.at[pl.ds(sid * RPT + z * EB, EB)])
        plsc.subcore_barrier()

        for j in range(8):
            hidx0[pl.ds(16 * j, 16)] = srcb[0, pl.ds(16 * j, 16)] + hh * NPAD
        pltpu.async_copy(h8h.at[hidx0], rows0, gsem0)
        pltpu.async_copy(exoh.at[pl.ds(row0 * EB, EB)], exb0, gsem0)

        @pl.loop(0, NBPT, step=2)
        def _(base):
            for i in range(2):
                b = base + i
                o = 1 - i
                pltpu.make_async_copy(h8h.at[hxl[i]], rwl[i], gsl[i]).wait()
                pltpu.make_async_copy(
                    exoh.at[pl.ds((row0 + b) * EB, EB)], exl[i],
                    gsl[i]).wait()

                @pl.when(b + 1 < NBPT)
                def _():
                    for j in range(8):
                        hxl[o][pl.ds(16 * j, 16)] = (
                            srcb[b + 1, pl.ds(16 * j, 16)] + hh * NPAD)
                    pltpu.async_copy(h8h.at[hxl[o]], rwl[o], gsl[o])
                    pltpu.async_copy(
                        exoh.at[pl.ds((row0 + b + 1) * EB, EB)], exl[o],
                        gsl[o])

                rows, exb = rwl[i], exl[i]

                @pl.loop(0, EB, unroll=4)
                def _(r):
                    exr = exb[r, :]
                    s = jnp.full(
                        (16,), jnp.squeeze(lax.slice(exr, (hh,), (hh + 1,))),
                        _F32)
                    for j in range(4):
                        sl = pl.ds(16 * j, 16)
                        rows[r, sl] = rows[r, sl] * s

                pltpu.sync_copy(rows, oacc.at[dstb.at[b]], add=True)

        plsc.subcore_barrier()
        pltpu.sync_copy(
            oacc.at[pl.ds(sid * RPT, RPT)],
            owh.at[pl.ds((cid * H + hh) * NPAD + sid * RPT, RPT)])
        plsc.subcore_barrier()


def _sc_gat(t1, t2, h8f, src2, dst2):
    mesh = plsc.VectorSubcoreMesh(core_axis_name="c", subcore_axis_name="s")
    run = pl.kernel(
        _sc_body,
        out_type=(
            jax.ShapeDtypeStruct((2 * H * NPAD, C), _F32),
            jax.ShapeDtypeStruct((2 * NPAD, 16), _F32),
            jax.ShapeDtypeStruct((EPAD, 16), _F32),
        ),
        mesh=mesh,
        compiler_params=pltpu.CompilerParams(use_tc_tiling_on_sc=False),
        scratch_types=(
            pltpu.VMEM((NBPT, EB), jnp.int32),
            pltpu.VMEM((NBPT, EB), jnp.int32),
            pltpu.VMEM((EB, 16), _F32),
            pltpu.VMEM((EB, 16), _F32),
            pltpu.VMEM((EB, 16), _F32),
            pltpu.VMEM((EB, 16), _F32),
            pltpu.VMEM((EB, 16), _F32),
            pltpu.VMEM((EB, 16), _F32),
            pltpu.VMEM((EB, C), _F32),
            pltpu.VMEM((EB, C), _F32),
            pltpu.VMEM((EB,), jnp.int32),
            pltpu.VMEM((EB,), jnp.int32),
            pltpu.VMEM((EB, C), _F32),
            pltpu.VMEM((EB, 16), _F32),
            pltpu.VMEM_SHARED((NPAD, C), _F32),
            pltpu.VMEM_SHARED((NPAD, 16), _F32),
            pltpu.SemaphoreType.DMA,
            pltpu.SemaphoreType.DMA,
        ),
    )
    return run(t1, t2, h8f, src2, dst2)


def _build_am(a_s, a_d):
    eye8 = jnp.eye(8, dtype=_F32)
    ms = (a_s[:, :, None] * eye8[:, None, :]).reshape(D, 8)
    md = (a_d[:, :, None] * eye8[:, None, :]).reshape(D, 8)
    ms2 = jnp.concatenate([ms, ms], axis=1)
    md2 = jnp.concatenate([md, md], axis=1)
    return jnp.concatenate([ms2, md2], axis=1)


def kernel(x, edge_index, batch, W1, a1s, a1d, b1, W2, a2s, a2d, b2,
           W3, a3s, a3d, b3, Wfc, bfc):
    xp = jnp.zeros((NPAD, x.shape[1]), _F32).at[:N].set(x)
    pad = jnp.full((EPAD - E,), N, jnp.int32)
    src2 = jnp.concatenate([edge_index[0].astype(jnp.int32), pad]
                           ).reshape(NTILES, NBPT, EB)
    dst2 = jnp.concatenate([edge_index[1].astype(jnp.int32), pad]
                           ).reshape(NTILES, NBPT, EB)
    batch2 = jnp.zeros((NPAD,), jnp.int32).at[:N].set(
        batch.astype(jnp.int32)).reshape(NRB, 1, RB)

    wfcp = jnp.zeros((C, 128), _F32).at[:, :NCLS].set(Wfc)
    bfcp = jnp.zeros((1, 128), _F32).at[0, :NCLS].set(bfc)

    h8, t1, t2 = _tc_layer1(xp, W1, _build_am(a1s, a1d))
    ow, den, _ = _sc_gat(t1, t2, h8.reshape(H * NPAD, C), src2, dst2)

    h8, t1, t2 = _tc_prep_layer(ow.reshape(2, H, NPAD, C),
                                den.reshape(2, NPAD, 16),
                                b1.reshape(1, D), W2, _build_am(a2s, a2d))
    ow, den, _ = _sc_gat(t1, t2, h8.reshape(H * NPAD, C), src2, dst2)

    h8, t1, t2 = _tc_prep_layer(ow.reshape(2, H, NPAD, C),
                                den.reshape(2, NPAD, 16),
                                b2.reshape(1, D), W3, _build_am(a3s, a3d))
    ow, den, _ = _sc_gat(t1, t2, h8.reshape(H * NPAD, C), src2, dst2)

    out = _tc_final(ow.reshape(2, H, NPAD, C), den.reshape(2, NPAD, 16),
                    b3.reshape(1, C), batch2, wfcp, bfcp)
    return out[:, :NCLS]


# speedup vs baseline: 18.2708x; 1.1552x over previous
import jax
import jax.numpy as jnp
from jax import lax
from jax.experimental import pallas as pl
from jax.experimental.pallas import tpu as pltpu
from jax.experimental.pallas import tpu_sc as plsc

N = 10000
NPAD = 10240
E = 320000
H = 8
C = 64
D = 512
NG = 64
NCLS = 16

NCORE = 2
NSUB = 16
NTILES = NCORE * NSUB
EB = 128
EB2 = 64
NBPT = 80
NBPT2 = 160
EPT = NBPT * EB
EPAD = EPT * NTILES
RB = 256
NRB = NPAD // RB
RPT = NPAD // NSUB

_F32 = jnp.float32
_I32 = jnp.int32


def _mm_tail(h, h4_ref, t1_ref, t2_ref, am_ref):
    t = jnp.dot(h, am_ref[...], preferred_element_type=_F32)
    t1_ref[...] = t[:, :16]
    t2_ref[...] = t[:, 16:]
    for cc in range(4):
        h4_ref[cc] = h[:, 128 * cc:128 * (cc + 1)]


def _layer1_body(x_ref, w_ref, am_ref, h4_ref, t1_ref, t2_ref):
    h = jnp.dot(x_ref[...], w_ref[...], preferred_element_type=_F32)
    _mm_tail(h, h4_ref, t1_ref, t2_ref, am_ref)


def _rec2(rec, cc):
    a = jnp.broadcast_to(rec[:, 2 * cc:2 * cc + 1], (RB, C))
    b = jnp.broadcast_to(rec[:, 2 * cc + 1:2 * cc + 2], (RB, C))
    return jnp.concatenate([a, b], axis=1)


def _prep_body(ow_ref, den_ref, b_ref, w_ref, am_ref,
               h4_ref, t1_ref, t2_ref, xin_ref):
    den = den_ref[0] + den_ref[1]
    rec = 1.0 / (den + 1e-16)
    for cc in range(4):
        o = ow_ref[0, cc] + ow_ref[1, cc]
        sl = slice(128 * cc, 128 * (cc + 1))
        xin_ref[:, sl] = jnp.maximum(
            o * _rec2(rec, cc) + b_ref[0, sl], 0.0)
    h = jnp.dot(xin_ref[...], w_ref[...], preferred_element_type=_F32)
    _mm_tail(h, h4_ref, t1_ref, t2_ref, am_ref)


def _layer_out_shapes():
    return (
        jax.ShapeDtypeStruct((4, NPAD, 128), _F32),
        jax.ShapeDtypeStruct((NPAD, 16), _F32),
        jax.ShapeDtypeStruct((NPAD, 16), _F32),
    )


def _layer_out_specs():
    return (
        pl.BlockSpec((4, RB, 128), lambda i: (0, i, 0)),
        pl.BlockSpec((RB, 16), lambda i: (i, 0)),
        pl.BlockSpec((RB, 16), lambda i: (i, 0)),
    )


def _tc_layer1(xp, w, am):
    k = xp.shape[1]
    return pl.pallas_call(
        _layer1_body,
        grid=(NRB,),
        in_specs=[
            pl.BlockSpec((RB, k), lambda i: (i, 0)),
            pl.BlockSpec((k, D), lambda i: (0, 0)),
            pl.BlockSpec((D, 32), lambda i: (0, 0)),
        ],
        out_specs=_layer_out_specs(),
        out_shape=_layer_out_shapes(),
    )(xp, w, am)


def _tc_prep_layer(ow, den, b, w, am):
    return pl.pallas_call(
        _prep_body,
        grid=(NRB,),
        in_specs=[
            pl.BlockSpec((2, 4, RB, 128), lambda i: (0, 0, i, 0)),
            pl.BlockSpec((2, RB, 16), lambda i: (0, i, 0)),
            pl.BlockSpec((1, D), lambda i: (0, 0)),
            pl.BlockSpec((D, D), lambda i: (0, 0)),
            pl.BlockSpec((D, 32), lambda i: (0, 0)),
        ],
        out_specs=_layer_out_specs(),
        out_shape=_layer_out_shapes(),
        scratch_shapes=[pltpu.VMEM((RB, D), _F32)],
    )(ow, den, b, w, am)


def _final_body(ow_ref, den_ref, b3_ref, batch_ref, wfc_ref, bfc_ref,
                out_ref, pooled_ref, cnt_ref):
    i = pl.program_id(0)

    @pl.when(i == 0)
    def _():
        pooled_ref[...] = jnp.zeros_like(pooled_ref)
        cnt_ref[...] = jnp.zeros_like(cnt_ref)

    den = den_ref[0] + den_ref[1]
    rec = 1.0 / (den + 1e-16)
    acc = jnp.zeros((RB, C), _F32)
    for cc in range(4):
        o = ow_ref[0, cc] + ow_ref[1, cc]
        acc = acc + o[:, :C] * rec[:, 2 * cc:2 * cc + 1]
        acc = acc + o[:, C:] * rec[:, 2 * cc + 1:2 * cc + 2]
    h3 = jnp.maximum(acc / 8.0 + b3_ref[0], 0.0)

    rows = i * RB + lax.broadcasted_iota(_I32, (NG, RB), 1)
    valid = rows < N
    grp = lax.broadcasted_iota(_I32, (NG, RB), 0)
    onehot = jnp.where((batch_ref[0, 0][None, :] == grp) & valid, 1.0, 0.0)
    pooled_ref[...] += jnp.dot(onehot, h3, preferred_element_type=_F32)
    cnt_ref[...] += jnp.broadcast_to(
        jnp.sum(onehot, axis=1, keepdims=True), (NG, C))

    @pl.when(i == NRB - 1)
    def _():
        pd = pooled_ref[...] / jnp.maximum(cnt_ref[...], 1.0)
        logits = jnp.dot(pd, wfc_ref[...], preferred_element_type=_F32)
        logits = logits + bfc_ref[0]
        col = lax.broadcasted_iota(_I32, (NG, 128), 1)
        masked = jnp.where(col < NCLS, logits, -1e30)
        m = jnp.max(masked, axis=1, keepdims=True)
        ex = jnp.where(col < NCLS, jnp.exp(masked - m), 0.0)
        lse = m + jnp.log(jnp.sum(ex, axis=1, keepdims=True))
        out_ref[...] = masked - lse


def _tc_final(ow, den, b3, batch2, wfcp, bfcp):
    return pl.pallas_call(
        _final_body,
        grid=(NRB,),
        in_specs=[
            pl.BlockSpec((2, 4, RB, 128), lambda i: (0, 0, i, 0)),
            pl.BlockSpec((2, RB, 16), lambda i: (0, i, 0)),
            pl.BlockSpec((1, C), lambda i: (0, 0)),
            pl.BlockSpec((1, 1, RB), lambda i: (i, 0, 0)),
            pl.BlockSpec((C, 128), lambda i: (0, 0)),
            pl.BlockSpec((1, 128), lambda i: (0, 0)),
        ],
        out_specs=pl.BlockSpec((NG, 128), lambda i: (0, 0)),
        out_shape=jax.ShapeDtypeStruct((NG, 128), _F32),
        scratch_shapes=[pltpu.VMEM((NG, C), _F32), pltpu.VMEM((NG, C), _F32)],
    )(ow, den, b3, batch2, wfcp, bfcp)


def _sc_body(t1h, t2h, h4h, srch, dsth, owh, denh, exoh,
             ts0, ts1, td0, td1, exb0, exb1, sp0, sp1, dp0, dp1,
             rows0, rows1, hix0, hix1, dx0, dx1, ex20, ex21,
             zb, zbd, oacc, dacc, gsem0, gsem1, isem0, isem1):
    cid = lax.axis_index("c")
    sid = lax.axis_index("s")
    tile = cid * NSUB + sid
    e0 = tile * EPT
    zero16 = jnp.zeros((16,), _F32)
    tsl, tdl = (ts0, ts1), (td0, td1)
    exl, spl, dpl = (exb0, exb1), (sp0, sp1), (dp0, dp1)
    rwl, hxl = (rows0, rows1), (hix0, hix1)
    dxl, ex2l = (dx0, dx1), (ex20, ex21)
    gsl, isl = (gsem0, gsem1), (isem0, isem1)

    @pl.loop(0, EB, unroll=4)
    def _(r):
        @pl.when(r < 32)
        def _():
            for j in range(8):
                zb[r, pl.ds(16 * j, 16)] = zero16
        zbd[r, :] = zero16

    for z in range(RPT // EB):
        pltpu.sync_copy(zbd, dacc.at[pl.ds(sid * RPT + z * EB, EB)])
    plsc.subcore_barrier()

    pltpu.async_copy(srch.at[pl.ds(e0, EB)], sp0, isem0)
    pltpu.async_copy(dsth.at[pl.ds(e0, EB)], dp0, isem0)
    pltpu.make_async_copy(srch.at[pl.ds(e0, EB)], sp0, isem0).wait()
    pltpu.make_async_copy(dsth.at[pl.ds(e0, EB)], dp0, isem0).wait()
    pltpu.async_copy(t1h.at[sp0], ts0, gsem0)
    pltpu.async_copy(t2h.at[dp0], td0, gsem0)

    @pl.loop(0, NBPT, step=2)
    def _(base):
        for i in range(2):
            b = base + i
            o = 1 - i

            @pl.when(b + 1 < NBPT)
            def _():
                nb = e0 + (b + 1) * EB
                pltpu.async_copy(srch.at[pl.ds(nb, EB)], spl[o], isl[o])
                pltpu.async_copy(dsth.at[pl.ds(nb, EB)], dpl[o], isl[o])
                pltpu.make_async_copy(
                    srch.at[pl.ds(nb, EB)], spl[o], isl[o]).wait()
                pltpu.make_async_copy(
                    dsth.at[pl.ds(nb, EB)], dpl[o], isl[o]).wait()
                pltpu.async_copy(t1h.at[spl[o]], tsl[o], gsl[o])
                pltpu.async_copy(t2h.at[dpl[o]], tdl[o], gsl[o])

            pltpu.make_async_copy(t1h.at[spl[i]], tsl[i], gsl[i]).wait()
            pltpu.make_async_copy(t2h.at[dpl[i]], tdl[i], gsl[i]).wait()
            ts, td, exb = tsl[i], tdl[i], exl[i]

            @pl.loop(0, EB, unroll=4)
            def _(r):
                v = ts[r, :] + td[r, :]
                v = jnp.where(v >= 0.0, v, 0.2 * v)
                exb[r, :] = jnp.exp(v)

            pltpu.sync_copy(exb, exoh.at[pl.ds(e0 + b * EB, EB)])
            pltpu.sync_copy(exb, dacc.at[dpl[i]], add=True)

    plsc.subcore_barrier()
    pltpu.sync_copy(dacc.at[pl.ds(sid * RPT, RPT)],
                    denh.at[pl.ds(cid * NPAD + sid * RPT, RPT)])

    for cc in range(4):
        for z in range(RPT // 32):
            pltpu.sync_copy(zb, oacc.at[pl.ds(sid * RPT + z * 32, 32)])
        plsc.subcore_barrier()

        pltpu.async_copy(srch.at[pl.ds(e0, EB2)], sp0.at[pl.ds(0, EB2)],
                         isem0)
        pltpu.make_async_copy(
            srch.at[pl.ds(e0, EB2)], sp0.at[pl.ds(0, EB2)], isem0).wait()
        for j in range(4):
            hix0[pl.ds(16 * j, 16)] = sp0[pl.ds(16 * j, 16)] + cc * NPAD
        pltpu.async_copy(h4h.at[hix0], rows0, gsem0)
        pltpu.async_copy(exoh.at[pl.ds(e0, EB2)], ex20, gsem0)
        pltpu.async_copy(dsth.at[pl.ds(e0, EB2)], dx0, isem0)

        @pl.loop(0, NBPT2, step=2)
        def _(base):
            for i in range(2):
                b = base + i
                o = 1 - i

                @pl.when(b + 1 < NBPT2)
                def _():
                    nb = e0 + (b + 1) * EB2
                    pltpu.async_copy(
                        srch.at[pl.ds(nb, EB2)], spl[o].at[pl.ds(0, EB2)],
                        isl[o])
                    pltpu.make_async_copy(
                        srch.at[pl.ds(nb, EB2)], spl[o].at[pl.ds(0, EB2)],
                        isl[o]).wait()
                    for j in range(4):
                        hxl[o][pl.ds(16 * j, 16)] = (
                            spl[o][pl.ds(16 * j, 16)] + cc * NPAD)
                    pltpu.async_copy(h4h.at[hxl[o]], rwl[o], gsl[o])
                    pltpu.async_copy(
                        exoh.at[pl.ds(nb, EB2)], ex2l[o], gsl[o])
                    pltpu.async_copy(
                        dsth.at[pl.ds(nb, EB2)], dxl[o], isl[o])

                pltpu.make_async_copy(h4h.at[hxl[i]], rwl[i], gsl[i]).wait()
                pltpu.make_async_copy(
                    exoh.at[pl.ds(e0 + b * EB2, EB2)], ex2l[i],
                    gsl[i]).wait()
                rows, ex2 = rwl[i], ex2l[i]

                @pl.loop(0, EB2, unroll=4)
                def _(r):
                    exr = ex2[r, :]
                    s0 = jnp.full(
                        (16,),
                        jnp.squeeze(lax.slice(exr, (2 * cc,), (2 * cc + 1,))),
                        _F32)
                    s1 = jnp.full(
                        (16,),
                        jnp.squeeze(
                            lax.slice(exr, (2 * cc + 1,), (2 * cc + 2,))),
                        _F32)
                    for j in range(8):
                        sl = pl.ds(16 * j, 16)
                        s = s0 if j < 4 else s1
                        rows[r, sl] = rows[r, sl] * s

                pltpu.make_async_copy(
                    dsth.at[pl.ds(e0 + b * EB2, EB2)], dxl[i],
                    isl[i]).wait()
                pltpu.sync_copy(rows, oacc.at[dxl[i]], add=True)

        plsc.subcore_barrier()
        pltpu.sync_copy(
            oacc.at[pl.ds(sid * RPT, RPT)],
            owh.at[pl.ds((cid * 4 + cc) * NPAD + sid * RPT, RPT)])
        plsc.subcore_barrier()


def _sc_gat(t1, t2, h4f, src, dst):
    mesh = plsc.VectorSubcoreMesh(core_axis_name="c", subcore_axis_name="s")
    run = pl.kernel(
        _sc_body,
        out_type=(
            jax.ShapeDtypeStruct((2 * 4 * NPAD, 128), _F32),
            jax.ShapeDtypeStruct((2 * NPAD, 16), _F32),
            jax.ShapeDtypeStruct((EPAD, 16), _F32),
        ),
        mesh=mesh,
        compiler_params=pltpu.CompilerParams(use_tc_tiling_on_sc=False),
        scratch_types=(
            pltpu.VMEM((EB, 16), _F32),
            pltpu.VMEM((EB, 16), _F32),
            pltpu.VMEM((EB, 16), _F32),
            pltpu.VMEM((EB, 16), _F32),
            pltpu.VMEM((EB, 16), _F32),
            pltpu.VMEM((EB, 16), _F32),
            pltpu.VMEM((EB,), _I32),
            pltpu.VMEM((EB,), _I32),
            pltpu.VMEM((EB,), _I32),
            pltpu.VMEM((EB,), _I32),
            pltpu.VMEM((EB2, 128), _F32),
            pltpu.VMEM((EB2, 128), _F32),
            pltpu.VMEM((EB2,), _I32),
            pltpu.VMEM((EB2,), _I32),
            pltpu.VMEM((EB2,), _I32),
            pltpu.VMEM((EB2,), _I32),
            pltpu.VMEM((EB2, 16), _F32),
            pltpu.VMEM((EB2, 16), _F32),
            pltpu.VMEM((32, 128), _F32),
            pltpu.VMEM((EB, 16), _F32),
            pltpu.VMEM_SHARED((NPAD, 128), _F32),
            pltpu.VMEM_SHARED((NPAD, 16), _F32),
            pltpu.SemaphoreType.DMA,
            pltpu.SemaphoreType.DMA,
            pltpu.SemaphoreType.DMA,
            pltpu.SemaphoreType.DMA,
        ),
    )
    return run(t1, t2, h4f, src, dst)


def _build_am(a_s, a_d):
    eye8 = jnp.eye(8, dtype=_F32)
    ms = (a_s[:, :, None] * eye8[:, None, :]).reshape(D, 8)
    md = (a_d[:, :, None] * eye8[:, None, :]).reshape(D, 8)
    ms2 = jnp.concatenate([ms, ms], axis=1)
    md2 = jnp.concatenate([md, md], axis=1)
    return jnp.concatenate([ms2, md2], axis=1)


def kernel(x, edge_index, batch, W1, a1s, a1d, b1, W2, a2s, a2d, b2,
           W3, a3s, a3d, b3, Wfc, bfc):
    xp = jnp.zeros((NPAD, x.shape[1]), _F32).at[:N].set(x)
    pad = jnp.full((EPAD - E,), N, _I32)
    src = jnp.concatenate([edge_index[0].astype(_I32), pad])
    dst = jnp.concatenate([edge_index[1].astype(_I32), pad])
    batch2 = jnp.zeros((NPAD,), _I32).at[:N].set(
        batch.astype(_I32)).reshape(NRB, 1, RB)

    wfcp = jnp.zeros((C, 128), _F32).at[:, :NCLS].set(Wfc)
    bfcp = jnp.zeros((1, 128), _F32).at[0, :NCLS].set(bfc)

    h4, t1, t2 = _tc_layer1(xp, W1, _build_am(a1s, a1d))
    ow, den, _ = _sc_gat(t1, t2, h4.reshape(4 * NPAD, 128), src, dst)

    h4, t1, t2 = _tc_prep_layer(ow.reshape(2, 4, NPAD, 128),
                                den.reshape(2, NPAD, 16),
                                b1.reshape(1, D), W2, _build_am(a2s, a2d))
    ow, den, _ = _sc_gat(t1, t2, h4.reshape(4 * NPAD, 128), src, dst)

    h4, t1, t2 = _tc_prep_layer(ow.reshape(2, 4, NPAD, 128),
                                den.reshape(2, NPAD, 16),
                                b2.reshape(1, D), W3, _build_am(a3s, a3d))
    ow, den, _ = _sc_gat(t1, t2, h4.reshape(4 * NPAD, 128), src, dst)

    out = _tc_final(ow.reshape(2, 4, NPAD, 128), den.reshape(2, NPAD, 16),
                    b3.reshape(1, C), batch2, wfcp, bfcp)
    return out[:, :NCLS]
